# baseline (speedup 1.0000x reference)
import functools

import jax
import jax.numpy as jnp
from jax import lax
from jax.experimental import pallas as pl
from jax.experimental.pallas import tpu as pltpu
from jax.experimental.pallas import tpu_sc as plsc

VOCAB = 100000
D_MODEL = 64
BATCH = 1024
HIST = 200

_NC = 2
_NS = 16
_NW = _NC * _NS
_BPW = BATCH // _NW
_CHUNKS = ((0, 104), (104, 96))


def _emb_pool_body(ids_hbm, table_hbm, out_hbm, idx_v, rows_v, acc_v, sem):
    wid = lax.axis_index("s") * _NC + lax.axis_index("c")
    base = wid * _BPW
    pltpu.sync_copy(ids_hbm.at[pl.ds(base, _BPW)], idx_v)

    def one_row(i, _):
        cps = [
            pltpu.async_copy(
                table_hbm.at[idx_v.at[i, pl.ds(off, n)]],
                rows_v.at[pl.ds(off, n)],
                sem,
            )
            for off, n in _CHUNKS
        ]
        for cp in cps:
            cp.wait()

        def term(r, accs):
            return tuple(
                accs[c] + rows_v[r, pl.ds(c * 16, 16)] for c in range(4)
            )

        zero = jnp.zeros((16,), jnp.float32)
        accs = lax.fori_loop(0, HIST, term, (zero, zero, zero, zero))
        scale = jnp.float32(1.0 / HIST)
        for c in range(4):
            acc_v[i, pl.ds(c * 16, 16)] = accs[c] * scale
        return 0

    lax.fori_loop(0, _BPW, one_row, 0)
    pltpu.sync_copy(acc_v, out_hbm.at[pl.ds(base, _BPW)])


@functools.cache
def _emb_pool_kernel():
    return functools.partial(
        pl.kernel,
        out_type=jax.ShapeDtypeStruct((BATCH, D_MODEL), jnp.float32),
        mesh=plsc.VectorSubcoreMesh(core_axis_name="c", subcore_axis_name="s"),
        scratch_types=[
            pltpu.VMEM((_BPW, HIST), jnp.int32),
            pltpu.VMEM((HIST, D_MODEL), jnp.float32),
            pltpu.VMEM((_BPW, D_MODEL), jnp.float32),
            pltpu.SemaphoreType.DMA,
        ],
        compiler_params=pltpu.CompilerParams(use_tc_tiling_on_sc=False),
    )(_emb_pool_body)


_VT = 1024
_NT = -(-VOCAB // _VT)


def _logits_loss_body(xt_ref, w_ref, b_ref, lab_ref, logits_ref, loss_ref,
                      m_scr, s_scr, ll_scr):
    j = pl.program_id(0)

    @pl.when(j == 0)
    def _init():
        m_scr[...] = jnp.full((1, BATCH), -jnp.inf, jnp.float32)
        s_scr[...] = jnp.zeros((1, BATCH), jnp.float32)
        ll_scr[...] = jnp.zeros((1, BATCH), jnp.float32)
        loss_ref[...] = jnp.zeros((1, 1), jnp.float32)

    logits = (
        lax.dot_general(
            w_ref[...], xt_ref[...], (((0,), (0,)), ((), ())),
            preferred_element_type=jnp.float32,
        )
        + b_ref[...]
    )
    logits_ref[...] = logits

    row = j * _VT + lax.broadcasted_iota(jnp.int32, (_VT, 1), 0)

    def update(lm):
        tmax = jnp.max(lm, axis=0, keepdims=True)
        m_new = jnp.maximum(m_scr[...], tmax)
        s_scr[...] = s_scr[...] * jnp.exp(m_scr[...] - m_new) + jnp.sum(
            jnp.exp(lm - m_new), axis=0, keepdims=True
        )
        m_scr[...] = m_new
        ll_scr[...] += jnp.sum(
            jnp.where(row == lab_ref[...], lm, 0.0), axis=0, keepdims=True
        )

    @pl.when(j < _NT - 1)
    def _full():
        update(logits)

    @pl.when(j == _NT - 1)
    def _last():
        update(jnp.where(row < VOCAB, logits, -jnp.inf))
        per_col = m_scr[...] + jnp.log(s_scr[...]) - ll_scr[...]
        loss_ref[...] = (jnp.sum(per_col) / jnp.float32(BATCH)).reshape(1, 1)


def _logits_loss(xt, w, b2, lab2):
    return pl.pallas_call(
        _logits_loss_body,
        grid=(_NT,),
        in_specs=[
            pl.BlockSpec((D_MODEL, BATCH), lambda j: (0, 0)),
            pl.BlockSpec((D_MODEL, _VT), lambda j: (0, j)),
            pl.BlockSpec((_VT, 1), lambda j: (j, 0)),
            pl.BlockSpec((1, BATCH), lambda j: (0, 0)),
        ],
        out_specs=[
            pl.BlockSpec((_VT, BATCH), lambda j: (j, 0)),
            pl.BlockSpec((1, 1), lambda j: (0, 0)),
        ],
        out_shape=[
            jax.ShapeDtypeStruct((VOCAB, BATCH), jnp.float32),
            jax.ShapeDtypeStruct((1, 1), jnp.float32),
        ],
        scratch_shapes=[
            pltpu.VMEM((1, BATCH), jnp.float32),
            pltpu.VMEM((1, BATCH), jnp.float32),
            pltpu.VMEM((1, BATCH), jnp.float32),
        ],
        compiler_params=pltpu.CompilerParams(
            dimension_semantics=("arbitrary",),
        ),
    )(xt, w, b2, lab2)


def kernel(input_ids, labels, emb_table, W, b):
    ids = input_ids.astype(jnp.int32)
    x = _emb_pool_kernel()(ids, emb_table)
    logits_t, loss = _logits_loss(
        x.T, W, b.reshape(VOCAB, 1), labels.astype(jnp.int32).reshape(1, BATCH)
    )
    return (loss[0, 0], logits_t.T)


# speedup vs baseline: 1.0558x; 1.0558x over previous
import functools

import jax
import jax.numpy as jnp
from jax import lax
from jax.experimental import pallas as pl
from jax.experimental.pallas import tpu as pltpu
from jax.experimental.pallas import tpu_sc as plsc

VOCAB = 100000
D_MODEL = 64
BATCH = 1024
HIST = 200

_NC = 2
_NS = 16
_NW = _NC * _NS
_BPW = BATCH // _NW
_CHUNKS = ((0, 104), (104, 96))


def _emb_pool_body(ids_hbm, table_hbm, out_hbm, idx_v, rows_v, acc_v, sem):
    wid = lax.axis_index("s") * _NC + lax.axis_index("c")
    base = wid * _BPW
    pltpu.sync_copy(ids_hbm.at[pl.ds(base, _BPW)], idx_v)

    def one_row(i, _):
        cps = [
            pltpu.async_copy(
                table_hbm.at[idx_v.at[i, pl.ds(off, n)]],
                rows_v.at[pl.ds(off, n)],
                sem,
            )
            for off, n in _CHUNKS
        ]
        for cp in cps:
            cp.wait()

        def term(r, accs):
            return tuple(
                accs[c] + rows_v[r, pl.ds(c * 16, 16)] for c in range(4)
            )

        zero = jnp.zeros((16,), jnp.float32)
        accs = lax.fori_loop(0, HIST, term, (zero, zero, zero, zero))
        scale = jnp.float32(1.0 / HIST)
        for c in range(4):
            acc_v[i, pl.ds(c * 16, 16)] = accs[c] * scale
        return 0

    lax.fori_loop(0, _BPW, one_row, 0)
    pltpu.sync_copy(acc_v, out_hbm.at[pl.ds(base, _BPW)])


@functools.cache
def _emb_pool_kernel():
    return functools.partial(
        pl.kernel,
        out_type=jax.ShapeDtypeStruct((BATCH, D_MODEL), jnp.float32),
        mesh=plsc.VectorSubcoreMesh(core_axis_name="c", subcore_axis_name="s"),
        scratch_types=[
            pltpu.VMEM((_BPW, HIST), jnp.int32),
            pltpu.VMEM((HIST, D_MODEL), jnp.float32),
            pltpu.VMEM((_BPW, D_MODEL), jnp.float32),
            pltpu.SemaphoreType.DMA,
        ],
        compiler_params=pltpu.CompilerParams(use_tc_tiling_on_sc=False),
    )(_emb_pool_body)


_VT = 2048
_NT = -(-VOCAB // _VT)


def _logits_loss_body(xt_ref, w_ref, b_ref, lab_ref, logits_ref, loss_ref,
                      m_scr, s_scr, ll_scr):
    j = pl.program_id(0)

    @pl.when(j == 0)
    def _init():
        m_scr[...] = jnp.full((1, BATCH), -jnp.inf, jnp.float32)
        s_scr[...] = jnp.zeros((1, BATCH), jnp.float32)
        ll_scr[...] = jnp.zeros((1, BATCH), jnp.float32)
        loss_ref[...] = jnp.zeros((1, 1), jnp.float32)

    logits = (
        lax.dot_general(
            w_ref[...], xt_ref[...], (((0,), (0,)), ((), ())),
            preferred_element_type=jnp.float32,
        )
        + b_ref[...]
    )
    logits_ref[...] = logits

    row = j * _VT + lax.broadcasted_iota(jnp.int32, (_VT, 1), 0)
    lm = jnp.where(row < VOCAB, logits, -jnp.inf)

    tmax = jnp.max(lm, axis=0, keepdims=True)
    m_new = jnp.maximum(m_scr[...], tmax)
    s_scr[...] = s_scr[...] * jnp.exp(m_scr[...] - m_new) + jnp.sum(
        jnp.exp(lm - m_new), axis=0, keepdims=True
    )
    m_scr[...] = m_new
    ll_scr[...] += jnp.sum(
        jnp.where(row == lab_ref[...], lm, 0.0), axis=0, keepdims=True
    )

    @pl.when(j == _NT - 1)
    def _fin():
        per_col = m_scr[...] + jnp.log(s_scr[...]) - ll_scr[...]
        loss_ref[...] = (jnp.sum(per_col) / jnp.float32(BATCH)).reshape(1, 1)


def _logits_loss(xt, w, b2, lab2):
    return pl.pallas_call(
        _logits_loss_body,
        grid=(_NT,),
        in_specs=[
            pl.BlockSpec((D_MODEL, BATCH), lambda j: (0, 0)),
            pl.BlockSpec((D_MODEL, _VT), lambda j: (0, j)),
            pl.BlockSpec((_VT, 1), lambda j: (j, 0)),
            pl.BlockSpec((1, BATCH), lambda j: (0, 0)),
        ],
        out_specs=[
            pl.BlockSpec((_VT, BATCH), lambda j: (j, 0)),
            pl.BlockSpec((1, 1), lambda j: (0, 0)),
        ],
        out_shape=[
            jax.ShapeDtypeStruct((VOCAB, BATCH), jnp.float32),
            jax.ShapeDtypeStruct((1, 1), jnp.float32),
        ],
        scratch_shapes=[
            pltpu.VMEM((1, BATCH), jnp.float32),
            pltpu.VMEM((1, BATCH), jnp.float32),
            pltpu.VMEM((1, BATCH), jnp.float32),
        ],
        compiler_params=pltpu.CompilerParams(
            dimension_semantics=("arbitrary",),
        ),
    )(xt, w, b2, lab2)


def kernel(input_ids, labels, emb_table, W, b):
    ids = input_ids.astype(jnp.int32)
    x = _emb_pool_kernel()(ids, emb_table)
    logits_t, loss = _logits_loss(
        x.T, W, b.reshape(VOCAB, 1), labels.astype(jnp.int32).reshape(1, BATCH)
    )
    return (loss[0, 0], logits_t.T)


# speedup vs baseline: 1.0780x; 1.0210x over previous
import functools

import jax
import jax.numpy as jnp
from jax import lax
from jax.experimental import pallas as pl
from jax.experimental.pallas import tpu as pltpu
from jax.experimental.pallas import tpu_sc as plsc

VOCAB = 100000
D_MODEL = 64
BATCH = 1024
HIST = 200

_NC = 2
_NS = 16
_NW = _NC * _NS
_BPW = BATCH // _NW
_CHUNKS = ((0, 104), (104, 96))


def _emb_pool_body(ids_hbm, table_hbm, out_hbm, idx_v, rows_v, acc_v,
                   sem0, sem1):
    wid = lax.axis_index("s") * _NC + lax.axis_index("c")
    base = wid * _BPW
    pltpu.sync_copy(ids_hbm.at[pl.ds(base, _BPW)], idx_v)
    sems = (sem0, sem1)

    def gather(i, buf, sem):
        i = jnp.minimum(i, _BPW - 1)
        for off, n in _CHUNKS:
            pltpu.async_copy(
                table_hbm.at[idx_v.at[i, pl.ds(off, n)]],
                rows_v.at[buf, pl.ds(off, n)],
                sem,
            )

    def wait(buf, sem):
        for off, n in _CHUNKS:
            pltpu.make_async_copy(
                table_hbm.at[idx_v.at[0, pl.ds(off, n)]],
                rows_v.at[buf, pl.ds(off, n)],
                sem,
            ).wait()

    def reduce_into(i, buf):
        def term(t, accs):
            r = 2 * t
            a = tuple(
                accs[c] + rows_v[buf, r, pl.ds(c * 16, 16)] for c in range(4)
            )
            return tuple(
                a[c] + rows_v[buf, r + 1, pl.ds(c * 16, 16)] for c in range(4)
            )

        zero = jnp.zeros((16,), jnp.float32)
        accs = lax.fori_loop(0, HIST // 2, term, (zero,) * 4)
        scale = jnp.float32(1.0 / HIST)
        for c in range(4):
            acc_v[i, pl.ds(c * 16, 16)] = accs[c] * scale

    gather(0, 0, sem0)

    def pair(g, _):
        gather(g + 1, 1, sem1)
        wait(0, sem0)
        reduce_into(g, 0)
        gather(g + 2, 0, sem0)
        wait(1, sem1)
        reduce_into(g + 1, 1)
        return 0

    lax.fori_loop(0, _BPW // 2, lambda g, c: pair(2 * g, c), 0)
    wait(0, sem0)
    pltpu.sync_copy(acc_v, out_hbm.at[pl.ds(base, _BPW)])


@functools.cache
def _emb_pool_kernel():
    return functools.partial(
        pl.kernel,
        out_type=jax.ShapeDtypeStruct((BATCH, D_MODEL), jnp.float32),
        mesh=plsc.VectorSubcoreMesh(core_axis_name="c", subcore_axis_name="s"),
        scratch_types=[
            pltpu.VMEM((_BPW, HIST), jnp.int32),
            pltpu.VMEM((2, HIST, D_MODEL), jnp.float32),
            pltpu.VMEM((_BPW, D_MODEL), jnp.float32),
            pltpu.SemaphoreType.DMA,
            pltpu.SemaphoreType.DMA,
        ],
        compiler_params=pltpu.CompilerParams(use_tc_tiling_on_sc=False),
    )(_emb_pool_body)


_VT = 1024
_NT = -(-VOCAB // _VT)


def _logits_loss_body(xt_ref, w_ref, b_ref, lab_ref, logits_ref, loss_ref,
                      m_scr, s_scr, ll_scr):
    j = pl.program_id(0)

    @pl.when(j == 0)
    def _init():
        m_scr[...] = jnp.full((1, BATCH), -jnp.inf, jnp.float32)
        s_scr[...] = jnp.zeros((1, BATCH), jnp.float32)
        ll_scr[...] = jnp.zeros((1, BATCH), jnp.float32)
        loss_ref[...] = jnp.zeros((1, 1), jnp.float32)

    logits = (
        lax.dot_general(
            w_ref[...], xt_ref[...], (((0,), (0,)), ((), ())),
            preferred_element_type=jnp.float32,
        )
        + b_ref[...]
    )
    logits_ref[...] = logits

    row = j * _VT + lax.broadcasted_iota(jnp.int32, (_VT, 1), 0)
    lm = jnp.where(row < VOCAB, logits, -jnp.inf)

    tmax = jnp.max(lm, axis=0, keepdims=True)
    m_new = jnp.maximum(m_scr[...], tmax)
    s_scr[...] = s_scr[...] * jnp.exp(m_scr[...] - m_new) + jnp.sum(
        jnp.exp(lm - m_new), axis=0, keepdims=True
    )
    m_scr[...] = m_new
    ll_scr[...] += jnp.sum(
        jnp.where(row == lab_ref[...], lm, 0.0), axis=0, keepdims=True
    )

    @pl.when(j == _NT - 1)
    def _fin():
        per_col = m_scr[...] + jnp.log(s_scr[...]) - ll_scr[...]
        loss_ref[...] = (jnp.sum(per_col) / jnp.float32(BATCH)).reshape(1, 1)


def _logits_loss(xt, w, b2, lab2):
    return pl.pallas_call(
        _logits_loss_body,
        grid=(_NT,),
        in_specs=[
            pl.BlockSpec((D_MODEL, BATCH), lambda j: (0, 0)),
            pl.BlockSpec((D_MODEL, _VT), lambda j: (0, j)),
            pl.BlockSpec((_VT, 1), lambda j: (j, 0)),
            pl.BlockSpec((1, BATCH), lambda j: (0, 0)),
        ],
        out_specs=[
            pl.BlockSpec((_VT, BATCH), lambda j: (j, 0)),
            pl.BlockSpec((1, 1), lambda j: (0, 0)),
        ],
        out_shape=[
            jax.ShapeDtypeStruct((VOCAB, BATCH), jnp.float32),
            jax.ShapeDtypeStruct((1, 1), jnp.float32),
        ],
        scratch_shapes=[
            pltpu.VMEM((1, BATCH), jnp.float32),
            pltpu.VMEM((1, BATCH), jnp.float32),
            pltpu.VMEM((1, BATCH), jnp.float32),
        ],
        compiler_params=pltpu.CompilerParams(
            dimension_semantics=("arbitrary",),
        ),
    )(xt, w, b2, lab2)


def kernel(input_ids, labels, emb_table, W, b):
    ids = input_ids.astype(jnp.int32)
    x = _emb_pool_kernel()(ids, emb_table)
    logits_t, loss = _logits_loss(
        x.T, W, b.reshape(VOCAB, 1), labels.astype(jnp.int32).reshape(1, BATCH)
    )
    return (loss[0, 0], logits_t.T)
